# Initial kernel scaffold; baseline (speedup 1.0000x reference)
import jax
import jax.numpy as jnp
from jax.experimental import pallas as pl


def kernel(x, router_logits, w_gate_up, w_down):
    raise NotImplementedError("write your pallas kernel here")


# speedup vs baseline: 1.0221x; 1.0221x over previous
import jax
import jax.numpy as jnp
from jax.experimental import pallas as pl
from jax.experimental.pallas import tpu as pltpu

E = 8
H = 1024
I = 2048
T = 2048
BT = 256
BI = 1024
NI = I // BI


def _gates_for_expert(logits, e):
    probs = jax.nn.softmax(logits, axis=-1)
    iota = jax.lax.broadcasted_iota(jnp.int32, logits.shape, 1)
    m0 = jnp.max(probs, axis=-1, keepdims=True)
    i0 = jnp.min(jnp.where(probs == m0, iota, E), axis=-1, keepdims=True)
    pmask = jnp.where(iota == i0, -jnp.inf, probs)
    m1 = jnp.max(pmask, axis=-1, keepdims=True)
    i1 = jnp.min(jnp.where(pmask == m1, iota, E), axis=-1, keepdims=True)
    denom = m0 + m1
    return jnp.where(i0 == e, m0 / denom,
                     jnp.where(i1 == e, m1 / denom, 0.0))


def _dense_body(x_ref, lg_ref, wg_ref, wu_ref, wd_ref, out_ref):
    e = pl.program_id(0)
    i = pl.program_id(1)
    t = pl.program_id(2)
    rows = pl.ds(t * BT, BT)
    xt = x_ref[rows, :]
    gate = _gates_for_expert(lg_ref[rows, :], e)

    g = jnp.dot(xt, wg_ref[0], preferred_element_type=jnp.float32)
    u = jnp.dot(xt, wu_ref[0], preferred_element_type=jnp.float32)
    act = g * jax.nn.sigmoid(g) * u
    y = jnp.dot(act, wd_ref[0], preferred_element_type=jnp.float32)
    contrib = gate * y

    first = jnp.logical_and(e == 0, i == 0)

    @pl.when(first)
    def _():
        out_ref[rows, :] = contrib

    @pl.when(jnp.logical_not(first))
    def _():
        out_ref[rows, :] += contrib


@jax.jit
def kernel(x, router_logits, w_gate_up, w_down):
    w_g = w_gate_up[:, :, :I]
    w_u = w_gate_up[:, :, I:]
    return pl.pallas_call(
        _dense_body,
        grid=(E, NI, T // BT),
        in_specs=[
            pl.BlockSpec((T, H), lambda e, i, t: (0, 0)),
            pl.BlockSpec((T, E), lambda e, i, t: (0, 0)),
            pl.BlockSpec((1, H, BI), lambda e, i, t: (e, 0, i)),
            pl.BlockSpec((1, H, BI), lambda e, i, t: (e, 0, i)),
            pl.BlockSpec((1, BI, H), lambda e, i, t: (e, i, 0)),
        ],
        out_specs=pl.BlockSpec((T, H), lambda e, i, t: (0, 0)),
        out_shape=jax.ShapeDtypeStruct((T, H), jnp.float32),
        compiler_params=pltpu.CompilerParams(
            dimension_semantics=("arbitrary", "arbitrary", "arbitrary"),
        ),
    )(x, router_logits, w_g, w_u, w_down)
